# Initial kernel scaffold; baseline (speedup 1.0000x reference)
import jax
import jax.numpy as jnp
from jax.experimental import pallas as pl


def kernel(x, adj, W1, b1, W_mu, b_mu, W_ls, b_ls, eps):
    raise NotImplementedError("write your pallas kernel here")


# speedup vs baseline: 19.2973x; 19.2973x over previous
import jax
import jax.numpy as jnp
from jax.experimental import pallas as pl
from jax.experimental.pallas import tpu as pltpu

N = 10000
NFEAT = 128
NHID = 64
NCLASS = 16

BR = 400
_CP = pltpu.CompilerParams(
    vmem_limit_bytes=64 * 1024 * 1024,
    dimension_semantics=("parallel",),
)


def _c1_kernel(x_ref, w1_ref, o_ref):
    o_ref[...] = jnp.dot(x_ref[...], w1_ref[...],
                         preferred_element_type=jnp.float32)


def _hidden_kernel(adj_ref, c1_ref, b1_ref, o_ref, abf_ref):
    a = adj_ref[...]
    acc = jnp.dot(a, c1_ref[...], preferred_element_type=jnp.float32)
    o_ref[...] = jax.nn.relu(acc + b1_ref[...])
    abf_ref[...] = a.astype(jnp.bfloat16)


def _z_kernel(adj_ref, h_ref, wc_ref, bc_ref, eps_ref, o_ref):
    h2 = jnp.dot(adj_ref[...], h_ref[...].astype(jnp.bfloat16),
                 preferred_element_type=jnp.float32)
    p = jnp.dot(h2, wc_ref[...], preferred_element_type=jnp.float32)
    p = p + bc_ref[...]
    mu = p[:, :NCLASS]
    ls = p[:, NCLASS:]
    o_ref[...] = eps_ref[...] * jnp.exp(ls) + mu


def _decode_kernel(zr_ref, z_ref, o_ref):
    logits = jax.lax.dot_general(
        zr_ref[...], z_ref[...],
        dimension_numbers=(((1,), (1,)), ((), ())),
        preferred_element_type=jnp.float32)
    o_ref[...] = jax.nn.sigmoid(logits)


def kernel(x, adj, W1, b1, W_mu, b_mu, W_ls, b_ls, eps):
    nb = N // BR

    c1 = pl.pallas_call(
        _c1_kernel,
        out_shape=jax.ShapeDtypeStruct((N, NHID), jnp.float32),
    )(x, W1)

    hidden, adj_bf = pl.pallas_call(
        _hidden_kernel,
        grid=(nb,),
        in_specs=[
            pl.BlockSpec((BR, N), lambda i: (i, 0)),
            pl.BlockSpec((N, NHID), lambda i: (0, 0)),
            pl.BlockSpec((1, NHID), lambda i: (0, 0)),
        ],
        out_specs=[
            pl.BlockSpec((BR, NHID), lambda i: (i, 0)),
            pl.BlockSpec((BR, N), lambda i: (i, 0)),
        ],
        out_shape=[
            jax.ShapeDtypeStruct((N, NHID), jnp.float32),
            jax.ShapeDtypeStruct((N, N), jnp.bfloat16),
        ],
        compiler_params=_CP,
    )(adj, c1, b1.reshape(1, NHID))

    wc = jnp.concatenate([W_mu, W_ls], axis=1)
    bc = jnp.concatenate([b_mu, b_ls]).reshape(1, 2 * NCLASS)
    z = pl.pallas_call(
        _z_kernel,
        grid=(nb,),
        in_specs=[
            pl.BlockSpec((BR, N), lambda i: (i, 0)),
            pl.BlockSpec((N, NHID), lambda i: (0, 0)),
            pl.BlockSpec((NHID, 2 * NCLASS), lambda i: (0, 0)),
            pl.BlockSpec((1, 2 * NCLASS), lambda i: (0, 0)),
            pl.BlockSpec((BR, NCLASS), lambda i: (i, 0)),
        ],
        out_specs=pl.BlockSpec((BR, NCLASS), lambda i: (i, 0)),
        out_shape=jax.ShapeDtypeStruct((N, NCLASS), jnp.float32),
        compiler_params=_CP,
    )(adj_bf, hidden, wc, bc, eps)

    a_pred = pl.pallas_call(
        _decode_kernel,
        grid=(nb,),
        in_specs=[
            pl.BlockSpec((BR, NCLASS), lambda i: (i, 0)),
            pl.BlockSpec((N, NCLASS), lambda i: (0, 0)),
        ],
        out_specs=pl.BlockSpec((BR, N), lambda i: (i, 0)),
        out_shape=jax.ShapeDtypeStruct((N, N), jnp.float32),
        compiler_params=_CP,
    )(z, z)

    return a_pred


# speedup vs baseline: 19.4503x; 1.0079x over previous
import jax
import jax.numpy as jnp
from jax.experimental import pallas as pl
from jax.experimental.pallas import tpu as pltpu

N = 10000
NFEAT = 128
NHID = 64
NCLASS = 16

BR = 200
_CP = pltpu.CompilerParams(
    vmem_limit_bytes=64 * 1024 * 1024,
    dimension_semantics=("parallel",),
)


def _c1_kernel(x_ref, w1_ref, o_ref):
    o_ref[...] = jnp.dot(x_ref[...], w1_ref[...],
                         preferred_element_type=jnp.float32)


def _hidden_kernel(adj_ref, c1_ref, b1_ref, o_ref):
    acc = jnp.dot(adj_ref[...], c1_ref[...],
                  preferred_element_type=jnp.float32)
    o_ref[...] = jax.nn.relu(acc + b1_ref[...])


def _z_kernel(adj_ref, h_ref, wc_ref, bc_ref, eps_ref, o_ref):
    h2 = jnp.dot(adj_ref[...], h_ref[...],
                 preferred_element_type=jnp.float32)
    p = jnp.dot(h2, wc_ref[...], preferred_element_type=jnp.float32)
    p = p + bc_ref[...]
    mu = p[:, :NCLASS]
    ls = p[:, NCLASS:]
    o_ref[...] = eps_ref[...] * jnp.exp(ls) + mu


def _decode_kernel(zr_ref, z_ref, o_ref):
    logits = jax.lax.dot_general(
        zr_ref[...], z_ref[...],
        dimension_numbers=(((1,), (1,)), ((), ())),
        preferred_element_type=jnp.float32)
    o_ref[...] = jax.nn.sigmoid(logits)


def kernel(x, adj, W1, b1, W_mu, b_mu, W_ls, b_ls, eps):
    nb = N // BR

    c1 = pl.pallas_call(
        _c1_kernel,
        out_shape=jax.ShapeDtypeStruct((N, NHID), jnp.float32),
    )(x, W1)

    hidden = pl.pallas_call(
        _hidden_kernel,
        grid=(nb,),
        in_specs=[
            pl.BlockSpec((BR, N), lambda i: (i, 0)),
            pl.BlockSpec((N, NHID), lambda i: (0, 0)),
            pl.BlockSpec((1, NHID), lambda i: (0, 0)),
        ],
        out_specs=pl.BlockSpec((BR, NHID), lambda i: (i, 0)),
        out_shape=jax.ShapeDtypeStruct((N, NHID), jnp.float32),
        compiler_params=_CP,
    )(adj, c1, b1.reshape(1, NHID))

    wc = jnp.concatenate([W_mu, W_ls], axis=1)
    bc = jnp.concatenate([b_mu, b_ls]).reshape(1, 2 * NCLASS)
    z = pl.pallas_call(
        _z_kernel,
        grid=(nb,),
        in_specs=[
            pl.BlockSpec((BR, N), lambda i: (i, 0)),
            pl.BlockSpec((N, NHID), lambda i: (0, 0)),
            pl.BlockSpec((NHID, 2 * NCLASS), lambda i: (0, 0)),
            pl.BlockSpec((1, 2 * NCLASS), lambda i: (0, 0)),
            pl.BlockSpec((BR, NCLASS), lambda i: (i, 0)),
        ],
        out_specs=pl.BlockSpec((BR, NCLASS), lambda i: (i, 0)),
        out_shape=jax.ShapeDtypeStruct((N, NCLASS), jnp.float32),
        compiler_params=_CP,
    )(adj, hidden, wc, bc, eps)

    a_pred = pl.pallas_call(
        _decode_kernel,
        grid=(nb,),
        in_specs=[
            pl.BlockSpec((BR, NCLASS), lambda i: (i, 0)),
            pl.BlockSpec((N, NCLASS), lambda i: (0, 0)),
        ],
        out_specs=pl.BlockSpec((BR, N), lambda i: (i, 0)),
        out_shape=jax.ShapeDtypeStruct((N, N), jnp.float32),
        compiler_params=_CP,
    )(z, z)

    return a_pred
